# baseline (speedup 1.0000x reference)
import jax
import jax.numpy as jnp
from jax.experimental import pallas as pl
from jax.experimental.pallas import tpu as pltpu

_BLOCK_ROWS = 10000


def _dot3(a_f32, bh, bl):
    ah = a_f32.astype(jnp.bfloat16)
    al = (a_f32 - ah.astype(jnp.float32)).astype(jnp.bfloat16)
    f32 = jnp.float32
    return (
        jnp.dot(ah, bh, preferred_element_type=f32)
        + jnp.dot(ah, bl, preferred_element_type=f32)
        + jnp.dot(al, bh, preferred_element_type=f32)
    )


def _mlp_body(x_ref, w1h_ref, w1l_ref, b1_ref, w2h_ref, w2l_ref, b2_ref, o_ref):
    h = _dot3(x_ref[...], w1h_ref[...], w1l_ref[...])
    h = jnp.maximum(h + b1_ref[...], 0.0)
    o = _dot3(h, w2h_ref[...], w2l_ref[...])
    o_ref[...] = o + b2_ref[...]


def _split_bf16(w):
    hi = w.astype(jnp.bfloat16)
    lo = (w - hi.astype(jnp.float32)).astype(jnp.bfloat16)
    return hi, lo


def kernel(x, W1, b1, W2, b2):
    n, d = x.shape
    h_dim = W1.shape[0]
    out_dim = W2.shape[0]
    w1h, w1l = _split_bf16(W1.T)
    w2h, w2l = _split_bf16(W2.T)
    b1r = b1.reshape(1, h_dim)
    b2r = b2.reshape(1, out_dim)
    grid = (n // _BLOCK_ROWS,)
    wspec = pl.BlockSpec((d, h_dim), lambda i: (0, 0))
    bspec = pl.BlockSpec((1, h_dim), lambda i: (0, 0))
    return pl.pallas_call(
        _mlp_body,
        grid=grid,
        in_specs=[
            pl.BlockSpec((_BLOCK_ROWS, d), lambda i: (i, 0)),
            wspec, wspec, bspec, wspec, wspec, bspec,
        ],
        out_specs=pl.BlockSpec((_BLOCK_ROWS, out_dim), lambda i: (i, 0)),
        out_shape=jax.ShapeDtypeStruct((n, out_dim), jnp.float32),
        compiler_params=pltpu.CompilerParams(
            dimension_semantics=("parallel",),
        ),
    )(x, w1h, w1l, b1r, w2h, w2l, b2r)


# speedup vs baseline: 1.2920x; 1.2920x over previous
import jax
import jax.numpy as jnp
from jax.experimental import pallas as pl
from jax.experimental.pallas import tpu as pltpu

_CHUNK = 2500
_NBUF = 4


def _body(x_hbm, w1t_hbm, b1_hbm, w2t_hbm, b2_hbm, o_hbm,
          x_buf, o_buf, w1t_v, b1_v, w2t_v, b2_v,
          in_sem, out_sem, w_sem):
    n = x_hbm.shape[0]
    nchunks = n // _CHUNK

    wcp1 = pltpu.make_async_copy(w1t_hbm, w1t_v, w_sem)
    wcp2 = pltpu.make_async_copy(w2t_hbm, w2t_v, w_sem)
    bcp1 = pltpu.make_async_copy(b1_hbm, b1_v, w_sem)
    bcp2 = pltpu.make_async_copy(b2_hbm, b2_v, w_sem)
    wcp1.start(); wcp2.start(); bcp1.start(); bcp2.start()

    def in_copy(i, slot):
        return pltpu.make_async_copy(
            x_hbm.at[pl.ds(i * _CHUNK, _CHUNK), :],
            x_buf.at[slot], in_sem.at[slot])

    def out_copy(i, slot):
        return pltpu.make_async_copy(
            o_buf.at[slot],
            o_hbm.at[pl.ds(i * _CHUNK, _CHUNK), :], out_sem.at[slot])

    for k in range(_NBUF):
        in_copy(k, k).start()

    wcp1.wait(); wcp2.wait(); bcp1.wait(); bcp2.wait()
    w1t = w1t_v[...]
    b1 = b1_v[...]
    w2t = w2t_v[...]
    b2 = b2_v[...]

    def step(i, _):
        slot = jax.lax.rem(i, _NBUF)
        in_copy(i, slot).wait()
        h = jnp.dot(x_buf[slot], w1t, preferred_element_type=jnp.float32)
        h = jnp.maximum(h + b1, 0.0)
        @pl.when(i >= _NBUF)
        def _():
            out_copy(i - _NBUF, slot).wait()
        o_buf[slot] = jnp.dot(h, w2t, preferred_element_type=jnp.float32) + b2
        out_copy(i, slot).start()
        @pl.when(i + _NBUF < nchunks)
        def _():
            in_copy(i + _NBUF, slot).start()
        return 0

    jax.lax.fori_loop(0, nchunks, step, 0)

    for k in range(_NBUF):
        i = nchunks - _NBUF + k
        out_copy(i, i % _NBUF).wait()


def kernel(x, W1, b1, W2, b2):
    n, d = x.shape
    h_dim = W1.shape[0]
    out_dim = W2.shape[0]
    w1t = W1.T
    w2t = W2.T
    b1r = b1.reshape(1, h_dim)
    b2r = b2.reshape(1, out_dim)
    anyspec = pl.BlockSpec(memory_space=pl.ANY)
    return pl.pallas_call(
        _body,
        in_specs=[anyspec] * 5,
        out_specs=anyspec,
        out_shape=jax.ShapeDtypeStruct((n, out_dim), jnp.float32),
        scratch_shapes=[
            pltpu.VMEM((_NBUF, _CHUNK, d), jnp.float32),
            pltpu.VMEM((_NBUF, _CHUNK, out_dim), jnp.float32),
            pltpu.VMEM((d, h_dim), jnp.float32),
            pltpu.VMEM((1, h_dim), jnp.float32),
            pltpu.VMEM((h_dim, out_dim), jnp.float32),
            pltpu.VMEM((1, out_dim), jnp.float32),
            pltpu.SemaphoreType.DMA((_NBUF,)),
            pltpu.SemaphoreType.DMA((_NBUF,)),
            pltpu.SemaphoreType.DMA,
        ],
    )(x, w1t, b1r, w2t, b2r)


# speedup vs baseline: 2.1415x; 1.6575x over previous
import jax
import jax.numpy as jnp
from jax.experimental import pallas as pl
from jax.experimental.pallas import tpu as pltpu

_BLOCK_ROWS = 10000


def _mlp_body(x_ref, w1t_hbm, b1_hbm, w2t_hbm, b2_hbm, o_ref,
              w1t_v, b1_v, w2t_v, b2_v, w_sem):
    @pl.when(pl.program_id(0) == 0)
    def _():
        for src, dst in ((w1t_hbm, w1t_v), (b1_hbm, b1_v),
                         (w2t_hbm, w2t_v), (b2_hbm, b2_v)):
            cp = pltpu.make_async_copy(src, dst, w_sem)
            cp.start()
            cp.wait()

    h = jnp.dot(x_ref[...], w1t_v[...], preferred_element_type=jnp.float32)
    h = jnp.maximum(h + b1_v[...], 0.0)
    o = jnp.dot(h, w2t_v[...], preferred_element_type=jnp.float32)
    o_ref[...] = o + b2_v[...]


def kernel(x, W1, b1, W2, b2):
    n, d = x.shape
    h_dim = W1.shape[0]
    out_dim = W2.shape[0]
    w1t = W1.T
    w2t = W2.T
    b1r = b1.reshape(1, h_dim)
    b2r = b2.reshape(1, out_dim)
    grid = (n // _BLOCK_ROWS,)
    anyspec = pl.BlockSpec(memory_space=pl.ANY)
    return pl.pallas_call(
        _mlp_body,
        grid=grid,
        in_specs=[
            pl.BlockSpec((_BLOCK_ROWS, d), lambda i: (i, 0)),
            anyspec, anyspec, anyspec, anyspec,
        ],
        out_specs=pl.BlockSpec((_BLOCK_ROWS, out_dim), lambda i: (i, 0)),
        out_shape=jax.ShapeDtypeStruct((n, out_dim), jnp.float32),
        scratch_shapes=[
            pltpu.VMEM((d, h_dim), jnp.float32),
            pltpu.VMEM((1, h_dim), jnp.float32),
            pltpu.VMEM((h_dim, out_dim), jnp.float32),
            pltpu.VMEM((1, out_dim), jnp.float32),
            pltpu.SemaphoreType.DMA,
        ],
        compiler_params=pltpu.CompilerParams(
            dimension_semantics=("arbitrary",),
        ),
    )(x, w1t, b1r, w2t, b2r)
